# baseline (speedup 1.0000x reference)
import functools

import jax
import jax.numpy as jnp
from jax import lax
from jax.experimental import pallas as pl
from jax.experimental.pallas import tpu as pltpu
from jax.experimental.pallas import tpu_sc as plsc


@functools.cache
def _make_sc_gather(T: int, B: int):
    info = plsc.get_sparse_core_info()
    NC, NS = 1, info.num_subcores
    NW = NC * NS
    assert B % (8 * NW) == 0
    b_per_w = B // NW
    mesh = plsc.VectorSubcoreMesh(
        core_axis_name="c", subcore_axis_name="s", num_cores=1)

    @functools.partial(
        pl.kernel,
        mesh=mesh,
        out_type=jax.ShapeDtypeStruct((B,), jnp.float32),
        scratch_types=[
            pltpu.VMEM_SHARED((T,), jnp.float32),
            pltpu.VMEM((b_per_w,), jnp.int32),
            pltpu.VMEM((b_per_w,), jnp.float32),
            pltpu.SemaphoreType.DMA,
            pltpu.SemaphoreType.DMA,
        ],
    )
    def sc_gather(arr_hbm, t_hbm, out_hbm, tab_sh, idx_v, val_v, s0, s1):
        wid = lax.axis_index("s") * NC + lax.axis_index("c")
        base = wid * b_per_w
        c1 = pltpu.async_copy(t_hbm.at[pl.ds(base, b_per_w)], idx_v, s1)

        @pl.when(wid == 0)
        def _():
            pltpu.sync_copy(arr_hbm, tab_sh)

        c1.wait()
        plsc.subcore_barrier()
        pltpu.async_copy(tab_sh.at[idx_v], val_v, s0).wait()
        pltpu.async_copy(val_v, out_hbm.at[pl.ds(base, b_per_w)], s1).wait()

    return sc_gather


def kernel(arr, t, x):
    B = t.shape[0]
    T = arr.shape[0]
    out = _make_sc_gather(T, B)(arr, t)
    return out.reshape((B,) + (1,) * (x.ndim - 1))
